# Initial kernel scaffold; baseline (speedup 1.0000x reference)
import jax
import jax.numpy as jnp
from jax.experimental import pallas as pl


def kernel(input_ids, unigram_table, ngram_table, W, b, norm_weight):
    raise NotImplementedError("write your pallas kernel here")


# speedup vs baseline: 5.2644x; 5.2644x over previous
import functools

import jax
import jax.numpy as jnp
from jax import lax
from jax.experimental import pallas as pl
from jax.experimental.pallas import tpu as pltpu
from jax.experimental.pallas import tpu_sc as plsc

NGRAM_VOCAB = 200000
HASH_MULT = 131

_NC = 2
_NS = 16
_NW = _NC * _NS

_K = 64


def _make_sc_gather(n_tokens, vocab, ngram_vocab, dim):
    pw = n_tokens // _NW
    nstep = pw // _K
    assert pw * _NW == n_tokens and nstep * _K == pw

    mesh = plsc.VectorSubcoreMesh(core_axis_name="c", subcore_axis_name="s")

    @functools.partial(
        pl.kernel,
        mesh=mesh,
        out_type=[
            jax.ShapeDtypeStruct((n_tokens, dim), jnp.float32),
            jax.ShapeDtypeStruct((n_tokens, dim), jnp.float32),
        ],
        scratch_types=[
            pltpu.VMEM((pw,), jnp.int32),
            pltpu.VMEM((pw,), jnp.int32),
            pltpu.VMEM((_K, dim), jnp.float32),
            pltpu.VMEM((_K, dim), jnp.float32),
            pltpu.SemaphoreType.DMA,
            pltpu.SemaphoreType.DMA,
            pltpu.SemaphoreType.DMA,
            pltpu.SemaphoreType.DMA,
        ],
    )
    def sc_gather(ids_hbm, prev_hbm, uni_hbm, ngr_hbm, uni_out, ngr_out,
                  ids_v, hsh_v, buf0, buf1, g0, g1, o0, o1):
        wid = lax.axis_index("s") * _NC + lax.axis_index("c")
        base = wid * pw

        pltpu.sync_copy(ids_hbm.at[pl.ds(base, pw)], ids_v)
        pltpu.sync_copy(prev_hbm.at[pl.ds(base, pw)], hsh_v)

        def hash_body(i, _):
            pv = hsh_v[pl.ds(i * 16, 16)]
            iv = ids_v[pl.ds(i * 16, 16)]
            hsh_v[pl.ds(i * 16, 16)] = lax.rem(pv * HASH_MULT + iv,
                                               ngram_vocab)
            return 0

        lax.fori_loop(0, pw // 16, hash_body, 0)

        def do_table(table_hbm, idx_ref, out_hbm):
            def step2(s2, _):
                s0 = s2 * 2
                pltpu.async_copy(
                    table_hbm.at[idx_ref.at[pl.ds(s0 * _K, _K)]], buf0, g0
                ).wait()
                cp0 = pltpu.async_copy(
                    buf0, out_hbm.at[pl.ds(base + s0 * _K, _K)], o0)
                pltpu.async_copy(
                    table_hbm.at[idx_ref.at[pl.ds((s0 + 1) * _K, _K)]],
                    buf1, g1
                ).wait()
                cp1 = pltpu.async_copy(
                    buf1, out_hbm.at[pl.ds(base + (s0 + 1) * _K, _K)], o1)
                cp0.wait()
                cp1.wait()
                return 0

            lax.fori_loop(0, nstep // 2, step2, 0)

        do_table(uni_hbm, ids_v, uni_out)
        do_table(ngr_hbm, hsh_v, ngr_out)

    return sc_gather


def _proj_body(uni_ref, ngr_ref, w1_ref, w2_ref, b_ref, nw_ref, out_ref):
    u = uni_ref[...]
    g = ngr_ref[...]
    acc = jnp.dot(u, w1_ref[...], preferred_element_type=jnp.float32)
    acc = acc + jnp.dot(g, w2_ref[...], preferred_element_type=jnp.float32)
    acc = acc + b_ref[...]
    var = jnp.mean(acc * acc, axis=-1, keepdims=True)
    out_ref[...] = acc * lax.rsqrt(var + 1e-6) * nw_ref[...]


def _tc_project(uni_rows, ngr_rows, w1, w2, b, nw, block_rows=512):
    n, d = uni_rows.shape
    grid = n // block_rows
    assert grid * block_rows == n
    return pl.pallas_call(
        _proj_body,
        grid=(grid,),
        in_specs=[
            pl.BlockSpec((block_rows, d), lambda i: (i, 0)),
            pl.BlockSpec((block_rows, d), lambda i: (i, 0)),
            pl.BlockSpec((d, d), lambda i: (0, 0)),
            pl.BlockSpec((d, d), lambda i: (0, 0)),
            pl.BlockSpec((1, d), lambda i: (0, 0)),
            pl.BlockSpec((1, d), lambda i: (0, 0)),
        ],
        out_specs=pl.BlockSpec((block_rows, d), lambda i: (i, 0)),
        out_shape=jax.ShapeDtypeStruct((n, d), jnp.float32),
    )(uni_rows, ngr_rows, w1, w2, b, nw)


def kernel(input_ids, unigram_table, ngram_table, W, b, norm_weight):
    bb, ss = input_ids.shape
    vocab, dim = unigram_table.shape
    ngram_vocab = ngram_table.shape[0]
    n = bb * ss

    ids = input_ids.reshape(n).astype(jnp.int32)
    prev = jnp.pad(input_ids, ((0, 0), (1, 0)))[:, :-1].reshape(n)
    prev = prev.astype(jnp.int32)

    sc_gather = _make_sc_gather(n, vocab, ngram_vocab, dim)
    uni_rows, ngr_rows = sc_gather(ids, prev, unigram_table, ngram_table)

    wt = W.T
    out = _tc_project(uni_rows, ngr_rows, wt[:dim], wt[dim:],
                      b.reshape(1, dim), norm_weight.reshape(1, dim))
    return out.reshape(bb, ss, dim)
